# Initial kernel scaffold; baseline (speedup 1.0000x reference)
import functools

import jax
import jax.numpy as jnp
from jax import lax
from jax.experimental import pallas as pl
from jax.experimental.pallas import tpu as pltpu
from jax.experimental.pallas import tpu_sc as plsc

D = 64
L = 16
NC, NS = 2, 16
NW = NC * NS
N = 4096 * 200
PER_W = N // NW
W = 512
NCHUNK = PER_W // W
GATHER = 128
NG = W // GATHER
POS_PAD = 256


def _comb_body(wt_ref, pos_ref, out_ref):
    out_ref[...] = wt_ref[...][:, None, :] + pos_ref[...][None, :, :]


def _build_comb(word_type_table, pos_table_padded):
    out3 = pl.pallas_call(
        _comb_body,
        out_shape=jax.ShapeDtypeStruct((8, POS_PAD, D), jnp.float32),
    )(word_type_table, pos_table_padded)
    return out3.reshape(8 * POS_PAD, D)


def _sc_body(seq_hbm, wt_hbm, pos_hbm, word_hbm, comb_hbm, out_hbm,
             seq_v, wt_v, pos_v, cidx_v, rows_w, rows_c, sem_w, sem_c):
    wid = lax.axis_index("s") * NC + lax.axis_index("c")
    base0 = wid * PER_W

    @pl.loop(0, NCHUNK)
    def _chunk(ci):
        base = base0 + ci * W
        pltpu.sync_copy(seq_hbm.at[pl.ds(base, W)], seq_v)
        pltpu.sync_copy(wt_hbm.at[pl.ds(base, W)], wt_v)
        pltpu.sync_copy(pos_hbm.at[pl.ds(base, W)], pos_v)
        for t in range(W // L):
            sl = pl.ds(t * L, L)
            cidx_v[sl] = wt_v[sl] * POS_PAD + pos_v[sl]
        copies = []
        for j in range(NG):
            sl = pl.ds(j * GATHER, GATHER)
            copies.append(
                pltpu.async_copy(word_hbm.at[seq_v.at[sl]], rows_w.at[sl], sem_w))
            copies.append(
                pltpu.async_copy(comb_hbm.at[cidx_v.at[sl]], rows_c.at[sl], sem_c))
        for cp in copies:
            cp.wait()

        @pl.loop(0, W)
        def _row(r):
            for c in range(D // L):
                sl2 = pl.ds(c * L, L)
                plsc.addupdate(rows_w.at[r, sl2], rows_c[r, sl2])

        pltpu.sync_copy(rows_w, out_hbm.at[pl.ds(base, W)])


@functools.partial(
    pl.kernel,
    out_type=jax.ShapeDtypeStruct((N, D), jnp.float32),
    mesh=plsc.VectorSubcoreMesh(core_axis_name="c", subcore_axis_name="s"),
    scratch_types=[
        pltpu.VMEM((W,), jnp.int32),
        pltpu.VMEM((W,), jnp.int32),
        pltpu.VMEM((W,), jnp.int32),
        pltpu.VMEM((W,), jnp.int32),
        pltpu.VMEM((W, D), jnp.float32),
        pltpu.VMEM((W, D), jnp.float32),
        pltpu.SemaphoreType.DMA,
        pltpu.SemaphoreType.DMA,
    ],
)
def _sc_lookup(seq_hbm, wt_hbm, pos_hbm, word_hbm, comb_hbm, out_hbm,
               seq_v, wt_v, pos_v, cidx_v, rows_w, rows_c, sem_w, sem_c):
    _sc_body(seq_hbm, wt_hbm, pos_hbm, word_hbm, comb_hbm, out_hbm,
             seq_v, wt_v, pos_v, cidx_v, rows_w, rows_c, sem_w, sem_c)


@jax.jit
def kernel(sequence, wtype, pos_enc, src_word_table, word_type_table,
           src_pos_table):
    B, Lseq = sequence.shape
    seq = sequence.reshape(-1).astype(jnp.int32)
    wt = wtype.reshape(-1).astype(jnp.int32)
    pos = pos_enc.reshape(-1).astype(jnp.int32)
    pos_padded = jnp.pad(src_pos_table,
                         ((0, POS_PAD - src_pos_table.shape[0]), (0, 0)))
    comb = _build_comb(word_type_table, pos_padded)
    out = _sc_lookup(seq, wt, pos, src_word_table, comb)
    return out.reshape(B, Lseq, D)


# speedup vs baseline: 8.2958x; 8.2958x over previous
import functools

import jax
import jax.numpy as jnp
from jax import lax
from jax.experimental import pallas as pl
from jax.experimental.pallas import tpu as pltpu
from jax.experimental.pallas import tpu_sc as plsc

D = 64
L = 16
NC, NS = 2, 16
NW = NC * NS
N = 4096 * 200
PER_W = N // NW
W = 512
NCHUNK = PER_W // W
GATHER = 128
NG = W // GATHER
POS_PAD = 256


def _comb_body(wt_ref, pos_ref, out_ref):
    out_ref[...] = wt_ref[...][:, None, :] + pos_ref[...][None, :, :]


def _build_comb(word_type_table, pos_table_padded):
    out3 = pl.pallas_call(
        _comb_body,
        out_shape=jax.ShapeDtypeStruct((8, POS_PAD, D), jnp.float32),
    )(word_type_table, pos_table_padded)
    return out3.reshape(8 * POS_PAD, D)


def _sc_body(seq_hbm, wt_hbm, pos_hbm, word_hbm, comb_hbm, out_hbm,
             seq_v, wt_v, pos_v, cidx_v, rows_w, rows_c, sem_w, sem_c):
    wid = lax.axis_index("s") * NC + lax.axis_index("c")
    base0 = wid * PER_W

    @pl.loop(0, NCHUNK)
    def _chunk(ci):
        base = base0 + ci * W
        pltpu.sync_copy(seq_hbm.at[pl.ds(base, W)], seq_v)
        pltpu.sync_copy(wt_hbm.at[pl.ds(base, W)], wt_v)
        pltpu.sync_copy(pos_hbm.at[pl.ds(base, W)], pos_v)
        for t in range(W // L):
            sl = pl.ds(t * L, L)
            cidx_v[sl] = wt_v[sl] * POS_PAD + pos_v[sl]
        copies = []
        for j in range(NG):
            sl = pl.ds(j * GATHER, GATHER)
            copies.append(
                pltpu.async_copy(word_hbm.at[seq_v.at[sl]], rows_w.at[sl], sem_w))
            copies.append(
                pltpu.async_copy(comb_hbm.at[cidx_v.at[sl]], rows_c.at[sl], sem_c))
        for cp in copies:
            cp.wait()

        @pl.loop(0, W)
        def _row(r):
            for c in range(D // L):
                sl2 = pl.ds(c * L, L)
                plsc.addupdate(rows_w.at[r, sl2], rows_c[r, sl2])

        pltpu.sync_copy(rows_w, out_hbm.at[pl.ds(base, W)])


@functools.partial(
    pl.kernel,
    out_type=jax.ShapeDtypeStruct((N, D), jnp.float32),
    mesh=plsc.VectorSubcoreMesh(core_axis_name="c", subcore_axis_name="s"),
    compiler_params=pltpu.CompilerParams(use_tc_tiling_on_sc=False),
    scratch_types=[
        pltpu.VMEM((W,), jnp.int32),
        pltpu.VMEM((W,), jnp.int32),
        pltpu.VMEM((W,), jnp.int32),
        pltpu.VMEM((W,), jnp.int32),
        pltpu.VMEM((W, D), jnp.float32),
        pltpu.VMEM((W, D), jnp.float32),
        pltpu.SemaphoreType.DMA,
        pltpu.SemaphoreType.DMA,
    ],
)
def _sc_lookup(seq_hbm, wt_hbm, pos_hbm, word_hbm, comb_hbm, out_hbm,
               seq_v, wt_v, pos_v, cidx_v, rows_w, rows_c, sem_w, sem_c):
    _sc_body(seq_hbm, wt_hbm, pos_hbm, word_hbm, comb_hbm, out_hbm,
             seq_v, wt_v, pos_v, cidx_v, rows_w, rows_c, sem_w, sem_c)


@jax.jit
def kernel(sequence, wtype, pos_enc, src_word_table, word_type_table,
           src_pos_table):
    B, Lseq = sequence.shape
    seq = sequence.reshape(-1).astype(jnp.int32)
    wt = wtype.reshape(-1).astype(jnp.int32)
    pos = pos_enc.reshape(-1).astype(jnp.int32)
    pos_padded = jnp.pad(src_pos_table,
                         ((0, POS_PAD - src_pos_table.shape[0]), (0, 0)))
    comb = _build_comb(word_type_table, pos_padded)
    out = _sc_lookup(seq, wt, pos, src_word_table, comb)
    return out.reshape(B, Lseq, D)


# speedup vs baseline: 8.2965x; 1.0001x over previous
import functools

import jax
import jax.numpy as jnp
from jax import lax
from jax.experimental import pallas as pl
from jax.experimental.pallas import tpu as pltpu
from jax.experimental.pallas import tpu_sc as plsc

D = 64
L = 16
NC, NS = 2, 16
NW = NC * NS
N = 4096 * 200
PER_W = N // NW
W = 512
NCHUNK = PER_W // W
GATHER = 128
NG = W // GATHER
POS_PAD = 256


def _comb_body(wt_ref, pos_ref, out_ref):
    out_ref[...] = wt_ref[...][:, None, :] + pos_ref[...][None, :, :]


def _build_comb(word_type_table, pos_table_padded):
    out3 = pl.pallas_call(
        _comb_body,
        out_shape=jax.ShapeDtypeStruct((8, POS_PAD, D), jnp.float32),
    )(word_type_table, pos_table_padded)
    return out3.reshape(8 * POS_PAD, D)


def _sc_body(seq_hbm, wt_hbm, pos_hbm, word_hbm, comb_hbm, out_hbm,
             seq_v, wt_v, pos_v, cidx_v, rows_w, rows_c, sem_w, sem_c):
    wid = lax.axis_index("s") * NC + lax.axis_index("c")
    base0 = wid * PER_W

    @pl.loop(0, NCHUNK)
    def _chunk(ci):
        base = base0 + ci * W
        pltpu.sync_copy(seq_hbm.at[pl.ds(base, W)], seq_v)
        pltpu.sync_copy(wt_hbm.at[pl.ds(base, W)], wt_v)
        pltpu.sync_copy(pos_hbm.at[pl.ds(base, W)], pos_v)
        for t in range(W // L):
            sl = pl.ds(t * L, L)
            cidx_v[sl] = wt_v[sl] * POS_PAD + pos_v[sl]
        copies = []
        for j in range(NG):
            sl = pl.ds(j * GATHER, GATHER)
            copies.append(
                pltpu.async_copy(word_hbm.at[seq_v.at[sl]], rows_w.at[sl], sem_w))
            copies.append(
                pltpu.async_copy(comb_hbm.at[cidx_v.at[sl]], rows_c.at[sl], sem_c))
        for cp in copies:
            cp.wait()

        @pl.loop(0, W)
        def _row(r):
            for c in range(D // L):
                sl2 = pl.ds(c * L, L)
                plsc.addupdate(rows_w.at[r, sl2], rows_c[r, sl2])

        pltpu.sync_copy(rows_w, out_hbm.at[pl.ds(base, W)])


@functools.partial(
    pl.kernel,
    out_type=jax.ShapeDtypeStruct((N, D), jnp.float32),
    mesh=plsc.VectorSubcoreMesh(core_axis_name="c", subcore_axis_name="s"),
    compiler_params=pltpu.CompilerParams(use_tc_tiling_on_sc=False),
    scratch_types=[
        pltpu.VMEM((W,), jnp.int32),
        pltpu.VMEM((W,), jnp.int32),
        pltpu.VMEM((W,), jnp.int32),
        pltpu.VMEM((W,), jnp.int32),
        pltpu.VMEM((W, D), jnp.float32),
        pltpu.VMEM((W, D), jnp.float32),
        pltpu.SemaphoreType.DMA,
        pltpu.SemaphoreType.DMA,
    ],
)
def _sc_lookup(seq_hbm, wt_hbm, pos_hbm, word_hbm, comb_hbm, out_hbm,
               seq_v, wt_v, pos_v, cidx_v, rows_w, rows_c, sem_w, sem_c):
    _sc_body(seq_hbm, wt_hbm, pos_hbm, word_hbm, comb_hbm, out_hbm,
             seq_v, wt_v, pos_v, cidx_v, rows_w, rows_c, sem_w, sem_c)


def _impl(sequence, wtype, pos_enc, src_word_table, word_type_table,
          src_pos_table):
    B, Lseq = sequence.shape
    seq = sequence.reshape(-1).astype(jnp.int32)
    wt = wtype.reshape(-1).astype(jnp.int32)
    pos = pos_enc.reshape(-1).astype(jnp.int32)
    pos_padded = jnp.pad(src_pos_table,
                         ((0, POS_PAD - src_pos_table.shape[0]), (0, 0)))
    comb = _build_comb(word_type_table, pos_padded)
    out = _sc_lookup(seq, wt, pos, src_word_table, comb)
    return out.reshape(B, Lseq, D)


_jitted = None


def kernel(sequence, wtype, pos_enc, src_word_table, word_type_table,
           src_pos_table):
    global _jitted
    if _jitted is None:
        from jax.experimental.layout import Format, Layout
        fmt = Format(Layout(major_to_minor=(0, 1, 2), tiling=()),
                     jax.sharding.SingleDeviceSharding(jax.devices()[0]))
        _jitted = jax.jit(_impl, out_shardings=fmt)
    return _jitted(sequence, wtype, pos_enc, src_word_table, word_type_table,
                   src_pos_table)
